# Initial kernel scaffold; baseline (speedup 1.0000x reference)
import jax
import jax.numpy as jnp
from jax.experimental import pallas as pl


def kernel(x, embed_table):
    raise NotImplementedError("write your pallas kernel here")


# speedup vs baseline: 2.1024x; 2.1024x over previous
import jax
import jax.numpy as jnp
from jax.experimental import pallas as pl


_B, _S, _D = 4, 4096, 1024
_E = 128
_SB = 512


def _concat_kernel(x_ref, tab_ref, out_ref):
    out_ref[:, :, :_D] = x_ref[...]
    out_ref[:, :, _D:] = tab_ref[...][None, :, :]


def kernel(x, embed_table):
    b, s, d = x.shape
    e = embed_table.shape[1]
    grid = (s // _SB, b)
    return pl.pallas_call(
        _concat_kernel,
        grid=grid,
        in_specs=[
            pl.BlockSpec((1, _SB, d), lambda i, j: (j, i, 0)),
            pl.BlockSpec((_SB, e), lambda i, j: (i, 0)),
        ],
        out_specs=pl.BlockSpec((1, _SB, d + e), lambda i, j: (j, i, 0)),
        out_shape=jax.ShapeDtypeStruct((b, s, d + e), x.dtype),
    )(x, embed_table)


# speedup vs baseline: 2.2945x; 1.0914x over previous
import jax
import jax.numpy as jnp
from jax.experimental import pallas as pl


_B, _S, _D = 4, 4096, 1024
_E = 128
_SB = 1024


def _concat_kernel(x_ref, tab_ref, out_ref):
    out_ref[:, :, :_D] = x_ref[...]
    out_ref[:, :, _D:] = tab_ref[...][None, :, :]


def kernel(x, embed_table):
    b, s, d = x.shape
    e = embed_table.shape[1]
    grid = (s // _SB, b)
    return pl.pallas_call(
        _concat_kernel,
        grid=grid,
        in_specs=[
            pl.BlockSpec((1, _SB, d), lambda i, j: (j, i, 0)),
            pl.BlockSpec((_SB, e), lambda i, j: (i, 0)),
        ],
        out_specs=pl.BlockSpec((1, _SB, d + e), lambda i, j: (j, i, 0)),
        out_shape=jax.ShapeDtypeStruct((b, s, d + e), x.dtype),
    )(x, embed_table)


# speedup vs baseline: 2.3825x; 1.0383x over previous
import jax
import jax.numpy as jnp
from jax.experimental import pallas as pl


_B, _S, _D = 4, 4096, 1024
_E = 128
_SB = 2048


def _concat_kernel(x_ref, tab_ref, out_ref):
    out_ref[:, :, :_D] = x_ref[...]
    out_ref[:, :, _D:] = tab_ref[...][None, :, :]


def kernel(x, embed_table):
    b, s, d = x.shape
    e = embed_table.shape[1]
    grid = (s // _SB, b)
    return pl.pallas_call(
        _concat_kernel,
        grid=grid,
        in_specs=[
            pl.BlockSpec((1, _SB, d), lambda i, j: (j, i, 0)),
            pl.BlockSpec((_SB, e), lambda i, j: (i, 0)),
        ],
        out_specs=pl.BlockSpec((1, _SB, d + e), lambda i, j: (j, i, 0)),
        out_shape=jax.ShapeDtypeStruct((b, s, d + e), x.dtype),
    )(x, embed_table)
